# baseline (speedup 1.0000x reference)
import jax
import jax.numpy as jnp
from jax import lax
from jax.experimental import pallas as pl
from jax.experimental.pallas import tpu as pltpu
from jax.experimental.pallas import tpu_sc as plsc
import functools


_B, _T, _M = 32, 576, 1024
_N = _B * _T

_TC_ROWS = 3072


def _argmax_block(x_ref, idx_ref):
    x = x_ref[...]
    m = jnp.max(x, axis=-1, keepdims=True)
    iota = jax.lax.broadcasted_iota(jnp.int32, x.shape, 1)
    cand = jnp.where(x == m, iota, _M)
    idx_ref[...] = jnp.min(cand, axis=-1).astype(jnp.int32)


def _tc_argmax(x2):
    return pl.pallas_call(
        _argmax_block,
        grid=(_N // _TC_ROWS,),
        in_specs=[pl.BlockSpec((_TC_ROWS, _M), lambda i: (i, 0))],
        out_specs=pl.BlockSpec((_TC_ROWS,), lambda i: (i,)),
        out_shape=jax.ShapeDtypeStruct((_N,), jnp.int32),
        compiler_params=pltpu.CompilerParams(
            dimension_semantics=("arbitrary",),
        ),
    )(x2)


_NC, _NS = 2, 16
_NW = _NC * _NS
_ROWS_PER_W = _N // _NW
_RB = 32
_NB = _ROWS_PER_W // _RB


def _sc_onehot_body(idx_hbm, out_hbm, idx_v, buf0, buf1, sem0, sem1):
    wid = lax.axis_index("s") * _NC + lax.axis_index("c")
    base = wid * _ROWS_PER_W
    pltpu.sync_copy(idx_hbm.at[pl.ds(base, _ROWS_PER_W)], idx_v)

    zero16 = jnp.zeros((16,), jnp.float32)
    one16 = jnp.ones((16,), jnp.float32)
    iota16 = lax.iota(jnp.int32, 16)
    bufs = (buf0, buf1)
    sems = (sem0, sem1)

    def zbody(i, _):
        buf0[pl.ds(i * 16, 16)] = zero16
        buf1[pl.ds(i * 16, 16)] = zero16
        return 0

    lax.fori_loop(0, _RB * _M // 16, zbody, 0)

    def scatter(buf, b, val16):
        for g in range(_RB // 16):
            col = idx_v[pl.ds(b * _RB + g * 16, 16)]
            pos = (iota16 + (g * 16)) * _M + col
            plsc.store_scatter(buf, [pos], val16)

    pending = [None, None]
    for b in range(_NB):
        k = b % 2
        buf, sem = bufs[k], sems[k]
        if pending[k] is not None:
            pending[k].wait()
            scatter(buf, b - 2, zero16)
        scatter(buf, b, one16)
        dst = out_hbm.at[pl.ds((base + b * _RB) * _M, _RB * _M)]
        pending[k] = pltpu.async_copy(buf, dst, sem)
    pending[(_NB - 2) % 2].wait()
    pending[(_NB - 1) % 2].wait()


_sc_onehot = functools.partial(
    pl.kernel,
    mesh=plsc.VectorSubcoreMesh(core_axis_name="c", subcore_axis_name="s"),
    out_type=jax.ShapeDtypeStruct((_N * _M,), jnp.float32),
    scratch_types=[
        pltpu.VMEM((_ROWS_PER_W,), jnp.int32),
        pltpu.VMEM((_RB * _M,), jnp.float32),
        pltpu.VMEM((_RB * _M,), jnp.float32),
        pltpu.SemaphoreType.DMA,
        pltpu.SemaphoreType.DMA,
    ],
    compiler_params=pltpu.CompilerParams(needs_layout_passes=False),
)(_sc_onehot_body)


def kernel(inputs):
    x2 = inputs.reshape(_N, _M)
    idx = _tc_argmax(x2)
    out_flat = _sc_onehot(idx)
    return out_flat.reshape(_B, _T, _M)


# speedup vs baseline: 1.9961x; 1.9961x over previous
import jax
import jax.numpy as jnp
from jax import lax
from jax.experimental import pallas as pl
from jax.experimental.pallas import tpu as pltpu
from jax.experimental.pallas import tpu_sc as plsc
import functools


_B, _T, _M = 32, 576, 1024
_N = _B * _T

_TC_ROWS = 3072


def _argmax_block(x_ref, idx_ref):
    x = x_ref[...]
    m = jnp.max(x, axis=-1, keepdims=True)
    iota = jax.lax.broadcasted_iota(jnp.int32, x.shape, 1)
    cand = jnp.where(x == m, iota, _M)
    idx_ref[...] = jnp.min(cand, axis=-1).astype(jnp.int32)


def _tc_argmax(x2):
    return pl.pallas_call(
        _argmax_block,
        grid=(_N // _TC_ROWS,),
        in_specs=[pl.BlockSpec((_TC_ROWS, _M), lambda i: (i, 0))],
        out_specs=pl.BlockSpec((_TC_ROWS,), lambda i: (i,)),
        out_shape=jax.ShapeDtypeStruct((_N,), jnp.int32),
        compiler_params=pltpu.CompilerParams(
            dimension_semantics=("arbitrary",),
        ),
    )(x2)


_NC, _NS = 2, 16
_NW = _NC * _NS
_ROWS_PER_W = _N // _NW
_RB = 32
_NB = _ROWS_PER_W // _RB


def _sc_onehot_body(idx_hbm, out_hbm, idx_v, buf0, buf1, sem0, sem1):
    wid = lax.axis_index("s") * _NC + lax.axis_index("c")
    base = wid * _ROWS_PER_W
    pltpu.sync_copy(idx_hbm.at[pl.ds(base, _ROWS_PER_W)], idx_v)

    zero16 = jnp.zeros((16,), jnp.float32)
    one16 = jnp.ones((16,), jnp.float32)
    iota16 = lax.iota(jnp.int32, 16)
    bufs = (buf0, buf1)
    sems = (sem0, sem1)

    def zbody(i, _):
        r = i >> 6
        c = (i & 63) * 16
        buf0[r, pl.ds(c, 16)] = zero16
        buf1[r, pl.ds(c, 16)] = zero16
        return 0

    lax.fori_loop(0, _RB * _M // 16, zbody, 0)

    def scatter(buf, b, val16):
        for g in range(_RB // 16):
            col = idx_v[pl.ds(b * _RB + g * 16, 16)]
            row = iota16 + (g * 16)
            plsc.store_scatter(buf, [row, col], val16)

    pending = [None, None]
    for b in range(_NB):
        k = b % 2
        buf, sem = bufs[k], sems[k]
        if pending[k] is not None:
            pending[k].wait()
            scatter(buf, b - 2, zero16)
        scatter(buf, b, one16)
        dst = out_hbm.at[pl.ds(base + b * _RB, _RB)]
        pending[k] = pltpu.async_copy(buf, dst, sem)
    pending[(_NB - 2) % 2].wait()
    pending[(_NB - 1) % 2].wait()


_sc_onehot = functools.partial(
    pl.kernel,
    mesh=plsc.VectorSubcoreMesh(core_axis_name="c", subcore_axis_name="s"),
    out_type=jax.ShapeDtypeStruct((_N, _M), jnp.float32),
    scratch_types=[
        pltpu.VMEM((_ROWS_PER_W,), jnp.int32),
        pltpu.VMEM((_RB, _M), jnp.float32),
        pltpu.VMEM((_RB, _M), jnp.float32),
        pltpu.SemaphoreType.DMA,
        pltpu.SemaphoreType.DMA,
    ],
    compiler_params=pltpu.CompilerParams(needs_layout_passes=False),
)(_sc_onehot_body)


def kernel(inputs):
    x2 = inputs.reshape(_N, _M)
    idx = _tc_argmax(x2)
    out = _sc_onehot(idx)
    return out.reshape(_B, _T, _M)
